# baseline (device time: 426409 ns/iter reference)
import jax
import jax.numpy as jnp
from jax import lax
from jax.experimental import pallas as pl
from jax.experimental.pallas import tpu as pltpu

N_DEV = 4
N_SLOT = 3
N_SEG = 2
N_HOP = N_DEV - 1
N_SUB = N_HOP * N_SEG


def kernel(A, B):
    A = A.astype(jnp.bfloat16)
    B = B.astype(jnp.bfloat16)
    m_per, k = A.shape
    n = B.shape[1]
    M = N_DEV * m_per
    m_half = m_per // N_SEG

    def body(a_ref, b_ref, z_ref, out_ref, comm_ref, acc_ref, send_sems,
             recv_sems, credit_sem, in_sem, out_sems):
        del z_ref
        my = lax.axis_index("i")
        left = (my - 1) % N_DEV
        right = (my + 1) % N_DEV

        in_copy = pltpu.make_async_copy(a_ref, comm_ref.at[0], in_sem)
        in_copy.start()

        barrier_sem = pltpu.get_barrier_semaphore()
        for nbr in (left, right):
            pl.semaphore_signal(
                barrier_sem, inc=1,
                device_id=(nbr,), device_id_type=pl.DeviceIdType.MESH,
            )
        pl.semaphore_wait(barrier_sem, 2)
        in_copy.wait()

        def sub_rdma(g):
            h = lax.div(g, N_SEG)
            s = lax.rem(g, N_SEG)
            rows = pl.ds(s * m_half, m_half)
            return pltpu.make_async_remote_copy(
                src_ref=comm_ref.at[lax.rem(h, N_SLOT), rows],
                dst_ref=comm_ref.at[lax.rem(h + 1, N_SLOT), rows],
                send_sem=send_sems.at[g],
                recv_sem=recv_sems.at[g],
                device_id=(right,),
                device_id_type=pl.DeviceIdType.MESH,
            )

        def step(g, carry):
            h = lax.div(g, N_SEG)
            s = lax.rem(g, N_SEG)
            slot = lax.rem(h, N_SLOT)

            @pl.when(h > 0)
            def _recv():
                sub_rdma((h - 1) * N_SEG + s).wait_recv()

            @pl.when(h < N_HOP)
            def _send():
                @pl.when(jnp.logical_and(h == N_HOP - 1, s == 0))
                def _():
                    pl.semaphore_wait(credit_sem, 1)
                sub_rdma(g).start()

            origin = lax.rem(my - h + N_DEV, N_DEV)
            aslot = lax.rem(g, 2)
            dst = out_ref.at[pl.ds(origin * m_per + s * m_half, m_half)]

            @pl.when(g >= 2)
            def _acc_free():
                pltpu.make_async_copy(
                    acc_ref.at[aslot], dst, out_sems.at[aslot]
                ).wait()

            acc_ref[aslot] = jnp.dot(
                comm_ref[slot, pl.ds(s * m_half, m_half), :],
                b_ref[...],
                preferred_element_type=jnp.float32,
            ).astype(jnp.bfloat16)
            pltpu.make_async_copy(
                acc_ref.at[aslot], dst, out_sems.at[aslot]
            ).start()

            @pl.when(g == 1)
            def _credit():
                sub_rdma(0).wait_send()
                sub_rdma(1).wait_send()
                pl.semaphore_signal(
                    credit_sem, inc=1,
                    device_id=(left,), device_id_type=pl.DeviceIdType.MESH,
                )
            return carry

        lax.fori_loop(0, N_DEV * N_SEG, step, 0, unroll=False)

        def drain(g, carry):
            sub_rdma(g).wait_send()
            return carry

        lax.fori_loop(2, N_SUB, drain, 0, unroll=False)

        def drain_out(t, carry):
            pltpu.make_async_copy(
                acc_ref.at[t], out_ref.at[pl.ds(0, m_half)], out_sems.at[t]
            ).wait()
            return carry

        lax.fori_loop(0, 2, drain_out, 0, unroll=False)

    return pl.pallas_call(
        body,
        out_shape=jax.ShapeDtypeStruct((M, n), jnp.bfloat16),
        in_specs=[
            pl.BlockSpec(memory_space=pl.ANY),
            pl.BlockSpec(memory_space=pltpu.VMEM),
            pl.BlockSpec(memory_space=pl.ANY),
        ],
        out_specs=pl.BlockSpec(memory_space=pl.ANY),
        input_output_aliases={2: 0},
        scratch_shapes=[
            pltpu.VMEM((N_SLOT, m_per, k), jnp.bfloat16),
            pltpu.VMEM((2, m_half, n), jnp.bfloat16),
            pltpu.SemaphoreType.DMA((N_SUB,)),
            pltpu.SemaphoreType.DMA((N_SUB,)),
            pltpu.SemaphoreType.REGULAR,
            pltpu.SemaphoreType.DMA,
            pltpu.SemaphoreType.DMA((2,)),
        ],
        compiler_params=pltpu.CompilerParams(
            collective_id=0,
            vmem_limit_bytes=60 * 1024 * 1024,
        ),
    )(A, B, jnp.zeros((M, n), jnp.bfloat16))


# device time: 392559 ns/iter; 1.0862x vs baseline; 1.0862x over previous
import jax
import jax.numpy as jnp
from jax import lax
from jax.experimental import pallas as pl
from jax.experimental.pallas import tpu as pltpu

N_DEV = 4
N_SLOT = 3
N_SEG = 4
N_HOP = N_DEV - 1
N_SUB = N_HOP * N_SEG


def kernel(A, B):
    A = A.astype(jnp.bfloat16)
    B = B.astype(jnp.bfloat16)
    m_per, k = A.shape
    n = B.shape[1]
    M = N_DEV * m_per
    m_half = m_per // N_SEG

    def body(a_ref, b_ref, out_ref, comm_ref, acc_ref, send_sems, recv_sems,
             credit_sem, in_sem, out_sems):
        my = lax.axis_index("i")
        left = (my - 1) % N_DEV
        right = (my + 1) % N_DEV

        in_copy = pltpu.make_async_copy(a_ref, comm_ref.at[0], in_sem)
        in_copy.start()

        barrier_sem = pltpu.get_barrier_semaphore()
        for nbr in (left, right):
            pl.semaphore_signal(
                barrier_sem, inc=1,
                device_id=(nbr,), device_id_type=pl.DeviceIdType.MESH,
            )
        pl.semaphore_wait(barrier_sem, 2)
        in_copy.wait()

        def sub_rdma(g):
            h = lax.div(g, N_SEG)
            s = lax.rem(g, N_SEG)
            rows = pl.ds(s * m_half, m_half)
            return pltpu.make_async_remote_copy(
                src_ref=comm_ref.at[lax.rem(h, N_SLOT), rows],
                dst_ref=comm_ref.at[lax.rem(h + 1, N_SLOT), rows],
                send_sem=send_sems.at[g],
                recv_sem=recv_sems.at[g],
                device_id=(right,),
                device_id_type=pl.DeviceIdType.MESH,
            )

        def step(g, carry):
            h = lax.div(g, N_SEG)
            s = lax.rem(g, N_SEG)
            slot = lax.rem(h, N_SLOT)

            @pl.when(h > 0)
            def _recv():
                sub_rdma((h - 1) * N_SEG + s).wait_recv()

            @pl.when(h < N_HOP)
            def _send():
                @pl.when(jnp.logical_and(h == N_HOP - 1, s == 0))
                def _():
                    pl.semaphore_wait(credit_sem, 1)
                sub_rdma(g).start()

            origin = lax.rem(my - h + N_DEV, N_DEV)
            aslot = lax.rem(g, 2)
            dst = out_ref.at[pl.ds(origin * m_per + s * m_half, m_half)]

            @pl.when(g >= 2)
            def _acc_free():
                pltpu.make_async_copy(
                    acc_ref.at[aslot], dst, out_sems.at[aslot]
                ).wait()

            acc_ref[aslot] = jnp.dot(
                comm_ref[slot, pl.ds(s * m_half, m_half), :],
                b_ref[...],
                preferred_element_type=jnp.float32,
            ).astype(jnp.bfloat16)
            pltpu.make_async_copy(
                acc_ref.at[aslot], dst, out_sems.at[aslot]
            ).start()

            @pl.when(g == N_SEG - 1)
            def _credit():
                def _ws(j, c):
                    sub_rdma(j).wait_send()
                    return c

                lax.fori_loop(0, N_SEG, _ws, 0, unroll=False)
                pl.semaphore_signal(
                    credit_sem, inc=1,
                    device_id=(left,), device_id_type=pl.DeviceIdType.MESH,
                )
            return carry

        lax.fori_loop(0, N_DEV * N_SEG, step, 0, unroll=False)

        def drain(g, carry):
            sub_rdma(g).wait_send()
            return carry

        lax.fori_loop(N_SEG, N_SUB, drain, 0, unroll=False)

        def drain_out(t, carry):
            pltpu.make_async_copy(
                acc_ref.at[t], out_ref.at[pl.ds(0, m_half)], out_sems.at[t]
            ).wait()
            return carry

        lax.fori_loop(0, 2, drain_out, 0, unroll=False)

    return pl.pallas_call(
        body,
        out_shape=jax.ShapeDtypeStruct((M, n), jnp.bfloat16),
        in_specs=[
            pl.BlockSpec(memory_space=pl.ANY),
            pl.BlockSpec(memory_space=pltpu.VMEM),
        ],
        out_specs=pl.BlockSpec(memory_space=pl.ANY),
        scratch_shapes=[
            pltpu.VMEM((N_SLOT, m_per, k), jnp.bfloat16),
            pltpu.VMEM((2, m_half, n), jnp.bfloat16),
            pltpu.SemaphoreType.DMA((N_SUB,)),
            pltpu.SemaphoreType.DMA((N_SUB,)),
            pltpu.SemaphoreType.REGULAR,
            pltpu.SemaphoreType.DMA,
            pltpu.SemaphoreType.DMA((2,)),
        ],
        compiler_params=pltpu.CompilerParams(
            collective_id=0,
            vmem_limit_bytes=60 * 1024 * 1024,
        ),
    )(A, B)
